# baseline (speedup 1.0000x reference)
import functools
import itertools

import numpy as np
import jax
import jax.numpy as jnp
from jax import lax
from jax.experimental import pallas as pl
from jax.experimental.pallas import tpu as pltpu
from jax.experimental.pallas import tpu_sc as plsc

_ORDER = 8
_T = 2**_ORDER - 1
_B = 16384
_NC, _NS = 2, 16
_NW = _NC * _NS
_L = 16

_SC_B = 6144
_TC_B = _B - _SC_B
_BPW = _SC_B // _NW
_TC_BLK = 1024
_SUB = _TC_BLK // 128


def _combo_masks():
    combos = []
    for i in range(1, _ORDER + 1):
        combos.extend(itertools.combinations(range(_ORDER), i))
    masks = np.zeros((_T,), dtype=np.int64)
    for t, c in enumerate(combos):
        for j in c:
            masks[t] |= 1 << j
    return masks


_MASKS = _combo_masks()

_PERM = np.zeros((256,), dtype=np.int32)
_PERM[_MASKS] = np.arange(_T, dtype=np.int32)
_PERM_PACKED = (
    (_PERM.reshape(64, 4).astype(np.int64) * (1 << (8 * np.arange(4, dtype=np.int64)))).sum(axis=1)
)
_PERM_PACKED = [int(x) - (1 << 32) if x >= (1 << 31) else int(x) for x in _PERM_PACKED]

_CODES = np.arange(256, dtype=np.int64)
_POPC = np.zeros((256, _T), dtype=np.int64)
_AND = _CODES[:, None] & _MASKS[None, :]
for _bit in range(8):
    _POPC += (_AND >> _bit) & 1
_SIGNS = (1.0 - 2.0 * (_POPC & 1)).astype(np.float32)
_POW2 = np.zeros((128,), dtype=np.float32)
_POW2[:_ORDER] = (1 << np.arange(_ORDER)).astype(np.float32)

_MESH = plsc.VectorSubcoreMesh(
    core_axis_name="c", subcore_axis_name="s", num_cores=_NC, num_subcores=_NS
)


@functools.partial(
    pl.kernel,
    mesh=_MESH,
    compiler_params=pltpu.CompilerParams(needs_layout_passes=False),
    out_type=jax.ShapeDtypeStruct((_SC_B,), jnp.float32),
    scratch_types=[
        pltpu.VMEM((_BPW, 128), jnp.int32),
        pltpu.VMEM((256,), jnp.float32),
        pltpu.VMEM((256,), jnp.float32),
        pltpu.VMEM((64,), jnp.int32),
        pltpu.VMEM((_BPW,), jnp.float32),
        pltpu.SemaphoreType.DMA,
    ],
)
def _kobe_sc(bits_hbm, vars_hbm, out_hbm, bits_v, vars_v, lut_v, packed_v, out_v, sem):
    wid = lax.axis_index("s") * _NC + lax.axis_index("c")
    base = wid * _BPW

    bits_dma = pltpu.async_copy(
        bits_hbm.at[pl.ds(base, _BPW), pl.ds(0, 128)], bits_v, sem
    )
    pltpu.sync_copy(vars_hbm, vars_v.at[pl.ds(0, _T)])

    lane = lax.iota(jnp.int32, _L)

    for g in range(4):
        acc = jnp.full((_L,), _PERM_PACKED[g * _L], jnp.int32)
        for k in range(1, _L):
            acc = jnp.where(lane == k, jnp.int32(_PERM_PACKED[g * _L + k]), acc)
        packed_v[pl.ds(g * _L, _L)] = acc
    shamt = (lane & 3) * 8
    for i in range(256 // _L):
        pidx = (lane >> 2) + (4 * i)
        pg = plsc.load_gather(packed_v, [pidx])
        idx = (pg >> shamt) & 255
        lut_v[pl.ds(i * _L, _L)] = plsc.load_gather(vars_v, [idx])
    v0 = lut_v[pl.ds(0, _L)]
    lut_v[pl.ds(0, _L)] = jnp.where(lane == 0, jnp.float32(0.0), v0)

    def inlane_body(it, _):
        lg = it // (256 // _L)
        i = it % (256 // _L)
        d = jnp.int32(1) << lg
        sign = (1 - 2 * ((lane >> lg) & 1)).astype(jnp.float32)
        pidx = (lane ^ d) + i * _L
        x = lut_v[pl.ds(i * _L, _L)]
        partner = plsc.load_gather(lut_v, [pidx])
        lut_v[pl.ds(i * _L, _L)] = partner + sign * x
        return _

    lax.fori_loop(0, 4 * (256 // _L), inlane_body, 0, unroll=2)

    for dr in (1, 2, 4, 8):
        for p in range(16):
            if p & dr:
                continue
            a = lut_v[pl.ds(p * _L, _L)]
            b = lut_v[pl.ds((p + dr) * _L, _L)]
            lut_v[pl.ds(p * _L, _L)] = a + b
            lut_v[pl.ds((p + dr) * _L, _L)] = a - b

    bits_dma.wait()

    def main_body(k, _):
        rows = lane + k * _L
        acc = plsc.load_gather(bits_v, [rows, jnp.zeros((_L,), jnp.int32)])
        for j in range(1, _ORDER):
            bj = plsc.load_gather(bits_v, [rows, jnp.full((_L,), j, jnp.int32)])
            acc = acc + (bj << j)
        out_v[pl.ds(k * _L, _L)] = plsc.load_gather(lut_v, [acc])
        return _

    lax.fori_loop(0, _BPW // _L, main_body, 0, unroll=2)

    pltpu.sync_copy(out_v, out_hbm.at[pl.ds(base, _BPW)])


def _tc_body(bits_ref, vars_ref, signs_ref, pow2_ref, out_ref):
    signs = signs_ref[...]
    pow2 = pow2_ref[:]
    lut = jnp.dot(signs, vars_ref[:], preferred_element_type=jnp.float32)
    bits = bits_ref[...].astype(jnp.float32)
    b3 = bits.reshape(_SUB, 128, 128)
    codes = lax.dot_general(
        b3, pow2, (((2,), (0,)), ((), ())), preferred_element_type=jnp.float32
    )
    codes_i = codes.astype(jnp.int32)
    onehot = (
        codes_i[..., None] == lax.broadcasted_iota(jnp.int32, (1, 1, 256), 2)
    ).astype(jnp.float32)
    out_ref[...] = lax.dot_general(
        onehot, lut, (((2,), (0,)), ((), ())), preferred_element_type=jnp.float32
    )


_kobe_tc = pl.pallas_call(
    _tc_body,
    grid=(_TC_B // _TC_BLK,),
    in_specs=[
        pl.BlockSpec((_TC_BLK, 128), lambda i: (i + _SC_B // _TC_BLK, 0)),
        pl.BlockSpec((_T,), lambda i: (0,)),
        pl.BlockSpec((256, _T), lambda i: (0, 0)),
        pl.BlockSpec((128,), lambda i: (0,)),
    ],
    out_specs=pl.BlockSpec((_SUB, 128), lambda i: (i, 0)),
    out_shape=jax.ShapeDtypeStruct((_TC_B // 128, 128), jnp.float32),
)


def kernel(bitstrings, variables):
    v = variables.astype(jnp.float32)
    sc_part = _kobe_sc(bitstrings, v)
    tc_part = _kobe_tc(
        bitstrings, v, jnp.asarray(_SIGNS), jnp.asarray(_POW2)
    )
    return jnp.concatenate([sc_part, tc_part.reshape(_TC_B)])


# speedup vs baseline: 1.0310x; 1.0310x over previous
import functools
import itertools

import numpy as np
import jax
import jax.numpy as jnp
from jax import lax
from jax.experimental import pallas as pl
from jax.experimental.pallas import tpu as pltpu
from jax.experimental.pallas import tpu_sc as plsc

_ORDER = 8
_T = 2**_ORDER - 1
_B = 16384
_NC, _NS = 2, 16
_NW = _NC * _NS
_L = 16

_SC_B = 10240
_TC_B = _B - _SC_B
_BPW = _SC_B // _NW
_TC_BLK = 1024
_SUB = _TC_BLK // 128


def _combo_masks():
    combos = []
    for i in range(1, _ORDER + 1):
        combos.extend(itertools.combinations(range(_ORDER), i))
    masks = np.zeros((_T,), dtype=np.int64)
    for t, c in enumerate(combos):
        for j in c:
            masks[t] |= 1 << j
    return masks


_MASKS = _combo_masks()

_PERM = np.zeros((256,), dtype=np.int32)
_PERM[_MASKS] = np.arange(_T, dtype=np.int32)
_PERM_PACKED = (
    (_PERM.reshape(64, 4).astype(np.int64) * (1 << (8 * np.arange(4, dtype=np.int64)))).sum(axis=1)
)
_PERM_PACKED = [int(x) - (1 << 32) if x >= (1 << 31) else int(x) for x in _PERM_PACKED]

_CODES = np.arange(256, dtype=np.int64)
_POPC = np.zeros((256, _T), dtype=np.int64)
_AND = _CODES[:, None] & _MASKS[None, :]
for _bit in range(8):
    _POPC += (_AND >> _bit) & 1
_SIGNS = (1.0 - 2.0 * (_POPC & 1)).astype(np.float32)
_POW2 = np.zeros((128,), dtype=np.float32)
_POW2[:_ORDER] = (1 << np.arange(_ORDER)).astype(np.float32)

_MESH = plsc.VectorSubcoreMesh(
    core_axis_name="c", subcore_axis_name="s", num_cores=_NC, num_subcores=_NS
)


@functools.partial(
    pl.kernel,
    mesh=_MESH,
    compiler_params=pltpu.CompilerParams(needs_layout_passes=False),
    out_type=jax.ShapeDtypeStruct((_SC_B,), jnp.float32),
    scratch_types=[
        pltpu.VMEM((_BPW, 128), jnp.int32),
        pltpu.VMEM((256,), jnp.float32),
        pltpu.VMEM((256,), jnp.float32),
        pltpu.VMEM((64,), jnp.int32),
        pltpu.VMEM((_BPW,), jnp.float32),
        pltpu.SemaphoreType.DMA,
    ],
)
def _kobe_sc(bits_hbm, vars_hbm, out_hbm, bits_v, vars_v, lut_v, packed_v, out_v, sem):
    wid = lax.axis_index("s") * _NC + lax.axis_index("c")
    base = wid * _BPW

    bits_dma = pltpu.async_copy(
        bits_hbm.at[pl.ds(base, _BPW), pl.ds(0, 128)], bits_v, sem
    )
    pltpu.sync_copy(vars_hbm, vars_v.at[pl.ds(0, _T)])

    lane = lax.iota(jnp.int32, _L)

    for g in range(4):
        acc = jnp.full((_L,), _PERM_PACKED[g * _L], jnp.int32)
        for k in range(1, _L):
            acc = jnp.where(lane == k, jnp.int32(_PERM_PACKED[g * _L + k]), acc)
        packed_v[pl.ds(g * _L, _L)] = acc
    shamt = (lane & 3) * 8
    for i in range(256 // _L):
        pidx = (lane >> 2) + (4 * i)
        pg = plsc.load_gather(packed_v, [pidx])
        idx = (pg >> shamt) & 255
        lut_v[pl.ds(i * _L, _L)] = plsc.load_gather(vars_v, [idx])
    v0 = lut_v[pl.ds(0, _L)]
    lut_v[pl.ds(0, _L)] = jnp.where(lane == 0, jnp.float32(0.0), v0)

    def inlane_body(it, _):
        lg = it // (256 // _L)
        i = it % (256 // _L)
        d = jnp.int32(1) << lg
        sign = (1 - 2 * ((lane >> lg) & 1)).astype(jnp.float32)
        pidx = (lane ^ d) + i * _L
        x = lut_v[pl.ds(i * _L, _L)]
        partner = plsc.load_gather(lut_v, [pidx])
        lut_v[pl.ds(i * _L, _L)] = partner + sign * x
        return _

    lax.fori_loop(0, 4 * (256 // _L), inlane_body, 0, unroll=2)

    for dr in (1, 2, 4, 8):
        for p in range(16):
            if p & dr:
                continue
            a = lut_v[pl.ds(p * _L, _L)]
            b = lut_v[pl.ds((p + dr) * _L, _L)]
            lut_v[pl.ds(p * _L, _L)] = a + b
            lut_v[pl.ds((p + dr) * _L, _L)] = a - b

    bits_dma.wait()

    def main_body(k, _):
        rows = lane + k * _L
        acc = plsc.load_gather(bits_v, [rows, jnp.zeros((_L,), jnp.int32)])
        for j in range(1, _ORDER):
            bj = plsc.load_gather(bits_v, [rows, jnp.full((_L,), j, jnp.int32)])
            acc = acc + (bj << j)
        out_v[pl.ds(k * _L, _L)] = plsc.load_gather(lut_v, [acc])
        return _

    lax.fori_loop(0, _BPW // _L, main_body, 0, unroll=2)

    pltpu.sync_copy(out_v, out_hbm.at[pl.ds(base, _BPW)])


def _tc_body(bits_ref, vars_ref, signs_ref, pow2_ref, out_ref):
    signs = signs_ref[...]
    pow2 = pow2_ref[:]
    lut = jnp.dot(signs, vars_ref[:], preferred_element_type=jnp.float32)
    bits = bits_ref[...].astype(jnp.float32)
    b3 = bits.reshape(_SUB, 128, 128)
    codes = lax.dot_general(
        b3, pow2, (((2,), (0,)), ((), ())), preferred_element_type=jnp.float32
    )
    codes_i = codes.astype(jnp.int32)
    onehot = (
        codes_i[..., None] == lax.broadcasted_iota(jnp.int32, (1, 1, 256), 2)
    ).astype(jnp.float32)
    out_ref[...] = lax.dot_general(
        onehot, lut, (((2,), (0,)), ((), ())), preferred_element_type=jnp.float32
    )


_kobe_tc = pl.pallas_call(
    _tc_body,
    grid=(_TC_B // _TC_BLK,),
    in_specs=[
        pl.BlockSpec((_TC_BLK, 128), lambda i: (i + _SC_B // _TC_BLK, 0)),
        pl.BlockSpec((_T,), lambda i: (0,)),
        pl.BlockSpec((256, _T), lambda i: (0, 0)),
        pl.BlockSpec((128,), lambda i: (0,)),
    ],
    out_specs=pl.BlockSpec((_SUB, 128), lambda i: (i, 0)),
    out_shape=jax.ShapeDtypeStruct((_TC_B // 128, 128), jnp.float32),
)


def kernel(bitstrings, variables):
    v = variables.astype(jnp.float32)
    sc_part = _kobe_sc(bitstrings, v)
    tc_part = _kobe_tc(
        bitstrings, v, jnp.asarray(_SIGNS), jnp.asarray(_POW2)
    )
    return jnp.concatenate([sc_part, tc_part.reshape(_TC_B)])


# speedup vs baseline: 1.0595x; 1.0277x over previous
import functools
import itertools

import numpy as np
import jax
import jax.numpy as jnp
from jax import lax
from jax.experimental import pallas as pl
from jax.experimental.pallas import tpu as pltpu
from jax.experimental.pallas import tpu_sc as plsc

_ORDER = 8
_T = 2**_ORDER - 1
_B = 16384
_NC, _NS = 2, 16
_NW = _NC * _NS
_L = 16

_SC_B = 8192
_TC_B = _B - _SC_B
_BPW = _SC_B // _NW
_TC_BLK = 2048
_SUB = _TC_BLK // 128


def _combo_masks():
    combos = []
    for i in range(1, _ORDER + 1):
        combos.extend(itertools.combinations(range(_ORDER), i))
    masks = np.zeros((_T,), dtype=np.int64)
    for t, c in enumerate(combos):
        for j in c:
            masks[t] |= 1 << j
    return masks


_MASKS = _combo_masks()

_PERM = np.zeros((256,), dtype=np.int32)
_PERM[_MASKS] = np.arange(_T, dtype=np.int32)
_PERM_PACKED = (
    (_PERM.reshape(64, 4).astype(np.int64) * (1 << (8 * np.arange(4, dtype=np.int64)))).sum(axis=1)
)
_PERM_PACKED = [int(x) - (1 << 32) if x >= (1 << 31) else int(x) for x in _PERM_PACKED]

_CODES = np.arange(256, dtype=np.int64)
_POPC = np.zeros((256, _T), dtype=np.int64)
_AND = _CODES[:, None] & _MASKS[None, :]
for _bit in range(8):
    _POPC += (_AND >> _bit) & 1
_SIGNS = (1.0 - 2.0 * (_POPC & 1)).astype(np.float32)
_POW2 = np.zeros((128,), dtype=np.float32)
_POW2[:_ORDER] = (1 << np.arange(_ORDER)).astype(np.float32)

_MESH = plsc.VectorSubcoreMesh(
    core_axis_name="c", subcore_axis_name="s", num_cores=_NC, num_subcores=_NS
)


@functools.partial(
    pl.kernel,
    mesh=_MESH,
    compiler_params=pltpu.CompilerParams(needs_layout_passes=False),
    out_type=jax.ShapeDtypeStruct((_SC_B,), jnp.float32),
    scratch_types=[
        pltpu.VMEM((_BPW, 128), jnp.int32),
        pltpu.VMEM((256,), jnp.float32),
        pltpu.VMEM((256,), jnp.float32),
        pltpu.VMEM((64,), jnp.int32),
        pltpu.VMEM((_BPW,), jnp.float32),
        pltpu.SemaphoreType.DMA,
    ],
)
def _kobe_sc(bits_hbm, vars_hbm, out_hbm, bits_v, vars_v, lut_v, packed_v, out_v, sem):
    wid = lax.axis_index("s") * _NC + lax.axis_index("c")
    base = wid * _BPW

    bits_dma = pltpu.async_copy(
        bits_hbm.at[pl.ds(base, _BPW), pl.ds(0, 128)], bits_v, sem
    )
    pltpu.sync_copy(vars_hbm, vars_v.at[pl.ds(0, _T)])

    lane = lax.iota(jnp.int32, _L)

    for g in range(4):
        acc = jnp.full((_L,), _PERM_PACKED[g * _L], jnp.int32)
        for k in range(1, _L):
            acc = jnp.where(lane == k, jnp.int32(_PERM_PACKED[g * _L + k]), acc)
        packed_v[pl.ds(g * _L, _L)] = acc
    shamt = (lane & 3) * 8
    for i in range(256 // _L):
        pidx = (lane >> 2) + (4 * i)
        pg = plsc.load_gather(packed_v, [pidx])
        idx = (pg >> shamt) & 255
        lut_v[pl.ds(i * _L, _L)] = plsc.load_gather(vars_v, [idx])
    v0 = lut_v[pl.ds(0, _L)]
    lut_v[pl.ds(0, _L)] = jnp.where(lane == 0, jnp.float32(0.0), v0)

    def inlane_body(it, _):
        lg = it // (256 // _L)
        i = it % (256 // _L)
        d = jnp.int32(1) << lg
        sign = (1 - 2 * ((lane >> lg) & 1)).astype(jnp.float32)
        pidx = (lane ^ d) + i * _L
        x = lut_v[pl.ds(i * _L, _L)]
        partner = plsc.load_gather(lut_v, [pidx])
        lut_v[pl.ds(i * _L, _L)] = partner + sign * x
        return _

    lax.fori_loop(0, 4 * (256 // _L), inlane_body, 0, unroll=2)

    for dr in (1, 2, 4, 8):
        for p in range(16):
            if p & dr:
                continue
            a = lut_v[pl.ds(p * _L, _L)]
            b = lut_v[pl.ds((p + dr) * _L, _L)]
            lut_v[pl.ds(p * _L, _L)] = a + b
            lut_v[pl.ds((p + dr) * _L, _L)] = a - b

    bits_dma.wait()

    def main_body(k, _):
        rows = lane + k * _L
        acc = plsc.load_gather(bits_v, [rows, jnp.zeros((_L,), jnp.int32)])
        for j in range(1, _ORDER):
            bj = plsc.load_gather(bits_v, [rows, jnp.full((_L,), j, jnp.int32)])
            acc = acc + (bj << j)
        out_v[pl.ds(k * _L, _L)] = plsc.load_gather(lut_v, [acc])
        return _

    lax.fori_loop(0, _BPW // _L, main_body, 0, unroll=2)

    pltpu.sync_copy(out_v, out_hbm.at[pl.ds(base, _BPW)])


def _tc_body(bits_ref, vars_ref, signs_ref, pow2_ref, out_ref):
    signs = signs_ref[...]
    pow2 = pow2_ref[:]
    lut = jnp.dot(signs, vars_ref[:], preferred_element_type=jnp.float32)
    bits = bits_ref[...].astype(jnp.float32)
    b3 = bits.reshape(_SUB, 128, 128)
    codes = lax.dot_general(
        b3, pow2, (((2,), (0,)), ((), ())), preferred_element_type=jnp.float32
    )
    codes_i = codes.astype(jnp.int32)
    onehot = (
        codes_i[..., None] == lax.broadcasted_iota(jnp.int32, (1, 1, 256), 2)
    ).astype(jnp.float32)
    out_ref[...] = lax.dot_general(
        onehot, lut, (((2,), (0,)), ((), ())), preferred_element_type=jnp.float32
    )


_kobe_tc = pl.pallas_call(
    _tc_body,
    grid=(_TC_B // _TC_BLK,),
    in_specs=[
        pl.BlockSpec((_TC_BLK, 128), lambda i: (i + _SC_B // _TC_BLK, 0)),
        pl.BlockSpec((_T,), lambda i: (0,)),
        pl.BlockSpec((256, _T), lambda i: (0, 0)),
        pl.BlockSpec((128,), lambda i: (0,)),
    ],
    out_specs=pl.BlockSpec((_SUB, 128), lambda i: (i, 0)),
    out_shape=jax.ShapeDtypeStruct((_TC_B // 128, 128), jnp.float32),
)


def kernel(bitstrings, variables):
    v = variables.astype(jnp.float32)
    sc_part = _kobe_sc(bitstrings, v)
    tc_part = _kobe_tc(
        bitstrings, v, jnp.asarray(_SIGNS), jnp.asarray(_POW2)
    )
    return jnp.concatenate([sc_part, tc_part.reshape(_TC_B)])
